# Initial kernel scaffold; baseline (speedup 1.0000x reference)
import jax
import jax.numpy as jnp
from jax.experimental import pallas as pl


def kernel(features, edge_index, W1, b1, W2, b2):
    raise NotImplementedError("write your pallas kernel here")


# speedup vs baseline: 8.7479x; 8.7479x over previous
import functools

import jax
import jax.numpy as jnp
from jax import lax
from jax.experimental import pallas as pl
from jax.experimental.pallas import tpu as pltpu
from jax.experimental.pallas import tpu_sc as plsc

N = 10000
E = 320000
NC = 2
NS = 16
NW = NC * NS
CHUNK = 128
EPW = E // NW
NCHUNK = -(-EPW // CHUNK)
EPW_PAD = NCHUNK * CHUNK
E_PAD = NW * EPW_PAD
RPT = 8 * (-(-N // (NS * 8)))
R = NS * RPT


def _seg_sum_sc(d):
  mesh = plsc.VectorSubcoreMesh(
      core_axis_name="c", subcore_axis_name="s", num_cores=NC,
      num_subcores=NS)

  @functools.partial(
      pl.kernel,
      out_type=jax.ShapeDtypeStruct((NC, R, d), jnp.float32),
      mesh=mesh,
      scratch_types=[
          pltpu.VMEM((NCHUNK, CHUNK), jnp.int32),
          pltpu.VMEM((NCHUNK, CHUNK), jnp.int32),
          pltpu.VMEM((CHUNK, d), jnp.float32),
          pltpu.VMEM_SHARED((R, d), jnp.float32),
          pltpu.SemaphoreType.DMA,
      ],
      compiler_params=pltpu.CompilerParams(use_tc_tiling_on_sc=False),
  )
  def k(p_hbm, src_hbm, dst_hbm, zero_hbm, out_hbm,
        src_v, dst_v, rows_v, acc_sh, sem):
    c = lax.axis_index("c")
    s = lax.axis_index("s")
    wid = s * NC + c
    pltpu.sync_copy(zero_hbm.at[pl.ds(s * RPT, RPT)],
                    acc_sh.at[pl.ds(s * RPT, RPT)])
    pltpu.sync_copy(src_hbm.at[wid], src_v)
    pltpu.sync_copy(dst_hbm.at[wid], dst_v)
    plsc.subcore_barrier()

    def body(j, carry):
      pltpu.async_copy(p_hbm.at[src_v.at[j]], rows_v, sem).wait()
      pltpu.sync_copy(rows_v, acc_sh.at[dst_v.at[j]], add=True)
      return carry

    lax.fori_loop(0, NCHUNK, body, 0)
    plsc.subcore_barrier()
    pltpu.sync_copy(acc_sh.at[pl.ds(s * RPT, RPT)],
                    out_hbm.at[c, pl.ds(s * RPT, RPT)])

  return k


def _mm_kernel(x_ref, w_ref, o_ref):
  o_ref[...] = jnp.dot(x_ref[...], w_ref[...],
                       preferred_element_type=jnp.float32)


def _fuse_relu_mm_kernel(a_ref, b_ref, bias_ref, w_ref, o_ref):
  h = jnp.maximum(a_ref[...] + b_ref[...] + bias_ref[...], 0.0)
  o_ref[...] = jnp.dot(h, w_ref[...], preferred_element_type=jnp.float32)


def _fuse_bias_kernel(a_ref, b_ref, bias_ref, o_ref):
  o_ref[...] = a_ref[...] + b_ref[...] + bias_ref[...]


def kernel(features, edge_index, W1, b1, W2, b2):
  d1 = W1.shape[1]
  d2 = W2.shape[1]

  src = edge_index[0].astype(jnp.int32)
  dst = edge_index[1].astype(jnp.int32)
  pad = E_PAD - E
  src_t = jnp.concatenate([src, jnp.zeros((pad,), jnp.int32)])
  src_t = src_t.reshape(NW, NCHUNK, CHUNK)
  dst_t = jnp.concatenate([dst, jnp.full((pad,), N, jnp.int32)])
  dst_t = dst_t.reshape(NW, NCHUNK, CHUNK)
  zeros1 = jnp.zeros((R, d1), jnp.float32)
  zeros2 = jnp.zeros((R, d2), jnp.float32)

  p1 = pl.pallas_call(
      _mm_kernel,
      out_shape=jax.ShapeDtypeStruct((N, d1), jnp.float32),
  )(features, W1)

  part1 = _seg_sum_sc(d1)(p1, src_t, dst_t, zeros1)

  p2 = pl.pallas_call(
      _fuse_relu_mm_kernel,
      out_shape=jax.ShapeDtypeStruct((N, d2), jnp.float32),
  )(part1[0, :N], part1[1, :N], b1.reshape(1, d1), W2)

  part2 = _seg_sum_sc(d2)(p2, src_t, dst_t, zeros2)

  out = pl.pallas_call(
      _fuse_bias_kernel,
      out_shape=jax.ShapeDtypeStruct((N, d2), jnp.float32),
  )(part2[0, :N], part2[1, :N], b2.reshape(1, d2))

  return out
